# Initial kernel scaffold; baseline (speedup 1.0000x reference)
import functools

import jax
import jax.numpy as jnp
from jax import lax
from jax.experimental import pallas as pl
from jax.experimental.pallas import tpu as pltpu
from jax.experimental.pallas import tpu_sc as plsc

EMBED_DIM = 64
ROWS_PER_DMA = 128
NSUB = 5
CHUNK = NSUB * ROWS_PER_DMA


def _sc_lookup(ids_flat, mask_flat, table, *, num_cores, num_subcores):
    n_workers = num_cores * num_subcores
    b = ids_flat.shape[0]
    rows_per_worker = b // n_workers
    chunks_per_worker = rows_per_worker // CHUNK
    mesh = plsc.VectorSubcoreMesh(core_axis_name="c", subcore_axis_name="s")

    @functools.partial(
        pl.kernel,
        mesh=mesh,
        out_type=jax.ShapeDtypeStruct((b, EMBED_DIM), jnp.float32),
        scratch_types=[
            pltpu.VMEM((CHUNK,), jnp.int32),
            pltpu.VMEM((CHUNK,), jnp.int32),
            pltpu.VMEM((NSUB, ROWS_PER_DMA), jnp.int32),
            pltpu.VMEM((CHUNK, EMBED_DIM), jnp.float32),
            pltpu.SemaphoreType.DMA,
        ],
    )
    def body(ids_hbm, mask_hbm, table_hbm, out_hbm, ids_v, mask_v, idx_v,
             rows_v, sem):
        wid = lax.axis_index("s") * num_cores + lax.axis_index("c")
        wbase = wid * rows_per_worker

        def chunk_body(g, carry):
            base = wbase + g * CHUNK
            pltpu.sync_copy(ids_hbm.at[pl.ds(base, CHUNK)], ids_v)
            pltpu.sync_copy(mask_hbm.at[pl.ds(base, CHUNK)], mask_v)
            for s in range(NSUB):
                for i in range(ROWS_PER_DMA // 16):
                    sl = pl.ds(s * ROWS_PER_DMA + i * 16, 16)
                    idx_v[s, pl.ds(i * 16, 16)] = ids_v[sl] * mask_v[sl]
            handles = [
                pltpu.async_copy(
                    table_hbm.at[idx_v.at[s]],
                    rows_v.at[pl.ds(s * ROWS_PER_DMA, ROWS_PER_DMA)],
                    sem,
                )
                for s in range(NSUB)
            ]
            for h in handles:
                h.wait()

            def row_body(r, c2):
                m = mask_v[r].astype(jnp.float32)
                for j in range(EMBED_DIM // 16):
                    sl2 = pl.ds(j * 16, 16)
                    rows_v[r, sl2] = rows_v[r, sl2] * m
                return c2

            lax.fori_loop(0, CHUNK, row_body, 0, unroll=4)
            pltpu.sync_copy(rows_v, out_hbm.at[pl.ds(base, CHUNK)])
            return carry

        lax.fori_loop(0, chunks_per_worker, chunk_body, 0)

    return body(ids_flat, mask_flat, table)


def kernel(input_ids, input_mask, embedding_table):
    batch, hist = input_ids.shape
    ids_flat = input_ids.reshape(-1).astype(jnp.int32)
    mask_flat = input_mask.reshape(-1).astype(jnp.int32)
    info = plsc.get_sparse_core_info()
    out = _sc_lookup(
        ids_flat,
        mask_flat,
        embedding_table,
        num_cores=info.num_cores,
        num_subcores=info.num_subcores,
    )
    return out.reshape(batch, hist, EMBED_DIM)


# speedup vs baseline: 11.9349x; 11.9349x over previous
import functools

import jax
import jax.numpy as jnp
from jax import lax
from jax.experimental import pallas as pl
from jax.experimental.pallas import tpu as pltpu
from jax.experimental.pallas import tpu_sc as plsc

EMBED_DIM = 64
ROWS_PER_DMA = 128
NSUB = 4
CHUNK = NSUB * ROWS_PER_DMA
NBUF = 3


def _sc_lookup(ids_flat, mask_flat, table, *, num_cores, num_subcores):
    n_workers = num_cores * num_subcores
    b = ids_flat.shape[0]
    rows_per_worker = b // n_workers
    n_chunks = rows_per_worker // CHUNK
    mesh = plsc.VectorSubcoreMesh(core_axis_name="c", subcore_axis_name="s")

    scratch = []
    for _ in range(NBUF):
        scratch += [
            pltpu.VMEM((CHUNK,), jnp.int32),
            pltpu.VMEM((CHUNK,), jnp.int32),
            pltpu.VMEM((NSUB, ROWS_PER_DMA), jnp.int32),
            pltpu.VMEM((CHUNK, EMBED_DIM), jnp.float32),
            pltpu.SemaphoreType.DMA,
            pltpu.SemaphoreType.DMA,
            pltpu.SemaphoreType.DMA,
        ]

    @functools.partial(
        pl.kernel,
        mesh=mesh,
        compiler_params=pltpu.CompilerParams(use_tc_tiling_on_sc=False),
        out_type=jax.ShapeDtypeStruct((b, EMBED_DIM), jnp.float32),
        scratch_types=scratch,
    )
    def body(ids_hbm, mask_hbm, table_hbm, out_hbm, *bufs):
        ids_v = [bufs[7 * i + 0] for i in range(NBUF)]
        mask_v = [bufs[7 * i + 1] for i in range(NBUF)]
        idx_v = [bufs[7 * i + 2] for i in range(NBUF)]
        rows_v = [bufs[7 * i + 3] for i in range(NBUF)]
        ssem = [bufs[7 * i + 4] for i in range(NBUF)]
        gsem = [bufs[7 * i + 5] for i in range(NBUF)]
        osem = [bufs[7 * i + 6] for i in range(NBUF)]

        wid = lax.axis_index("s") * num_cores + lax.axis_index("c")
        wbase = wid * rows_per_worker
        iota = lax.iota(jnp.int32, 16)

        def chunk_base(g):
            return wbase + g * CHUNK

        def stage(nb, g):
            base = chunk_base(g)
            pltpu.async_copy(ids_hbm.at[pl.ds(base, CHUNK)], ids_v[nb],
                             ssem[nb])
            pltpu.async_copy(mask_hbm.at[pl.ds(base, CHUNK)], mask_v[nb],
                             ssem[nb])

        def stage_wait(nb, g):
            base = chunk_base(g)
            pltpu.make_async_copy(ids_hbm.at[pl.ds(base, CHUNK)], ids_v[nb],
                                  ssem[nb]).wait()
            pltpu.make_async_copy(mask_hbm.at[pl.ds(base, CHUNK)], mask_v[nb],
                                  ssem[nb]).wait()

        def out_wait(nb, g_prev):
            base = chunk_base(g_prev)
            pltpu.make_async_copy(rows_v[nb], out_hbm.at[pl.ds(base, CHUNK)],
                                  osem[nb]).wait()

        def prep(nb, g, wait_out):
            stage_wait(nb, g)
            base = chunk_base(g)
            for s in range(NSUB):
                for i in range(ROWS_PER_DMA // 16):
                    off = s * ROWS_PER_DMA + i * 16
                    sl = pl.ds(off, 16)
                    m = mask_v[nb][sl]
                    pos = (base + off) + iota
                    idx_v[nb][s, pl.ds(i * 16, 16)] = (
                        pos + m * (ids_v[nb][sl] - pos))
            if wait_out:
                out_wait(nb, g - NBUF)
            for s in range(NSUB):
                pltpu.async_copy(
                    table_hbm.at[idx_v[nb].at[s]],
                    rows_v[nb].at[pl.ds(s * ROWS_PER_DMA, ROWS_PER_DMA)],
                    gsem[nb],
                )

        def finish(nb, g):
            base = chunk_base(g)
            for s in range(NSUB):
                pltpu.make_async_copy(
                    table_hbm.at[idx_v[nb].at[s]],
                    rows_v[nb].at[pl.ds(s * ROWS_PER_DMA, ROWS_PER_DMA)],
                    gsem[nb],
                ).wait()

            def grp_body(t, c2):
                r0 = t * 16
                mv = mask_v[nb][pl.ds(r0, 16)].astype(jnp.float32)
                for i in range(16):
                    m = mv[i]
                    for j in range(EMBED_DIM // 16):
                        sl2 = pl.ds(j * 16, 16)
                        rows_v[nb][r0 + i, sl2] = rows_v[nb][r0 + i, sl2] * m
                return c2

            lax.fori_loop(0, CHUNK // 16, grp_body, 0)
            pltpu.async_copy(rows_v[nb], out_hbm.at[pl.ds(base, CHUNK)],
                             osem[nb])

        n_main = (n_chunks - NBUF - 1) // NBUF * NBUF

        for g in range(NBUF):
            stage(g, g)
        prep(0, 0, False)
        prep(1, 1, False)

        def main_body(t, carry):
            g0 = t * NBUF
            for k in range(NBUF):
                g = g0 + k
                finish(k, g)
                stage(k, g + NBUF)
                nb2 = (k + 2) % NBUF
                prep(nb2, g + 2, True)
            return carry

        for k in range(NBUF):
            finish(k, k)
            stage(k, k + NBUF)
            prep((k + 2) % NBUF, k + 2, k + 2 >= NBUF)
        lax.fori_loop(1, n_main // NBUF, main_body, 0)

        for g in range(n_main, n_chunks):
            nb = g % NBUF
            finish(nb, g)
            if g + NBUF < n_chunks:
                stage(nb, g + NBUF)
            if g + 2 < n_chunks:
                prep((g + 2) % NBUF, g + 2, g + 2 >= NBUF)
        for g in range(n_chunks - NBUF, n_chunks):
            out_wait(g % NBUF, g)

    return body(ids_flat, mask_flat, table)


def kernel(input_ids, input_mask, embedding_table):
    batch, hist = input_ids.shape
    ids_flat = input_ids.reshape(-1).astype(jnp.int32)
    mask_flat = input_mask.reshape(-1).astype(jnp.int32)
    info = plsc.get_sparse_core_info()
    out = _sc_lookup(
        ids_flat,
        mask_flat,
        embedding_table,
        num_cores=info.num_cores,
        num_subcores=info.num_subcores,
    )
    return out.reshape(batch, hist, EMBED_DIM)


# speedup vs baseline: 14.5046x; 1.2153x over previous
import functools

import jax
import jax.numpy as jnp
from jax import lax
from jax.experimental import pallas as pl
from jax.experimental.pallas import tpu as pltpu
from jax.experimental.pallas import tpu_sc as plsc

EMBED_DIM = 64
ROWS_PER_DMA = 128
NSUB = 4
CHUNK = NSUB * ROWS_PER_DMA
NBUF = 3


def _sc_lookup(ids_flat, mask_flat, table, *, num_cores, num_subcores):
    n_workers = num_cores * num_subcores
    b = ids_flat.shape[0]
    rows_per_worker = b // n_workers
    n_chunks = rows_per_worker // CHUNK
    mesh = plsc.VectorSubcoreMesh(core_axis_name="c", subcore_axis_name="s")

    scratch = []
    for _ in range(NBUF):
        scratch += [
            pltpu.VMEM((CHUNK,), jnp.int32),
            pltpu.VMEM((CHUNK,), jnp.int32),
            pltpu.VMEM((NSUB, ROWS_PER_DMA), jnp.int32),
            pltpu.VMEM((CHUNK, EMBED_DIM), jnp.float32),
            pltpu.SemaphoreType.DMA,
            pltpu.SemaphoreType.DMA,
            pltpu.SemaphoreType.DMA,
        ]

    @functools.partial(
        pl.kernel,
        mesh=mesh,
        compiler_params=pltpu.CompilerParams(use_tc_tiling_on_sc=False),
        out_type=jax.ShapeDtypeStruct((b, EMBED_DIM), jnp.float32),
        scratch_types=scratch,
    )
    def body(ids_hbm, mask_hbm, table_hbm, out_hbm, *bufs):
        ids_v = [bufs[7 * i + 0] for i in range(NBUF)]
        mask_v = [bufs[7 * i + 1] for i in range(NBUF)]
        idx_v = [bufs[7 * i + 2] for i in range(NBUF)]
        rows_v = [bufs[7 * i + 3] for i in range(NBUF)]
        ssem = [bufs[7 * i + 4] for i in range(NBUF)]
        gsem = [bufs[7 * i + 5] for i in range(NBUF)]
        osem = [bufs[7 * i + 6] for i in range(NBUF)]

        wid = lax.axis_index("s") * num_cores + lax.axis_index("c")
        wbase = wid * rows_per_worker
        iota = lax.iota(jnp.int32, 16)

        def chunk_base(g):
            return wbase + g * CHUNK

        def stage(nb, g):
            base = chunk_base(g)
            pltpu.async_copy(ids_hbm.at[pl.ds(base, CHUNK)], ids_v[nb],
                             ssem[nb])
            pltpu.async_copy(mask_hbm.at[pl.ds(base, CHUNK)], mask_v[nb],
                             ssem[nb])

        def stage_wait(nb, g):
            base = chunk_base(g)
            pltpu.make_async_copy(ids_hbm.at[pl.ds(base, CHUNK)], ids_v[nb],
                                  ssem[nb]).wait()
            pltpu.make_async_copy(mask_hbm.at[pl.ds(base, CHUNK)], mask_v[nb],
                                  ssem[nb]).wait()

        def out_wait(nb, g_prev):
            base = chunk_base(g_prev)
            pltpu.make_async_copy(rows_v[nb], out_hbm.at[pl.ds(base, CHUNK)],
                                  osem[nb]).wait()

        def prep(nb, g, wait_out):
            stage_wait(nb, g)
            base = chunk_base(g)
            for s in range(NSUB):
                for i in range(ROWS_PER_DMA // 16):
                    off = s * ROWS_PER_DMA + i * 16
                    sl = pl.ds(off, 16)
                    m = mask_v[nb][sl]
                    pos = (base + off) + iota
                    idx_v[nb][s, pl.ds(i * 16, 16)] = (
                        pos + m * (ids_v[nb][sl] - pos))
            if wait_out:
                out_wait(nb, g - NBUF)
            for s in range(NSUB):
                pltpu.async_copy(
                    table_hbm.at[idx_v[nb].at[s]],
                    rows_v[nb].at[pl.ds(s * ROWS_PER_DMA, ROWS_PER_DMA)],
                    gsem[nb],
                )

        def finish(nb, g):
            base = chunk_base(g)
            for s in range(NSUB):
                pltpu.make_async_copy(
                    table_hbm.at[idx_v[nb].at[s]],
                    rows_v[nb].at[pl.ds(s * ROWS_PER_DMA, ROWS_PER_DMA)],
                    gsem[nb],
                ).wait()

            @plsc.parallel_loop(0, CHUNK // 16, 1, unroll=2)
            def _(t):
                r0 = t * 16
                mv = mask_v[nb][pl.ds(r0, 16)].astype(jnp.float32)
                for i in range(16):
                    m = mv[i]
                    for j in range(EMBED_DIM // 16):
                        sl2 = pl.ds(j * 16, 16)
                        rows_v[nb][r0 + i, sl2] = rows_v[nb][r0 + i, sl2] * m
            pltpu.async_copy(rows_v[nb], out_hbm.at[pl.ds(base, CHUNK)],
                             osem[nb])

        n_main = (n_chunks - NBUF - 1) // NBUF * NBUF

        for g in range(NBUF):
            stage(g, g)
        prep(0, 0, False)
        prep(1, 1, False)

        def main_body(t, carry):
            g0 = t * NBUF
            for k in range(NBUF):
                g = g0 + k
                finish(k, g)
                stage(k, g + NBUF)
                nb2 = (k + 2) % NBUF
                prep(nb2, g + 2, True)
            return carry

        for k in range(NBUF):
            finish(k, k)
            stage(k, k + NBUF)
            prep((k + 2) % NBUF, k + 2, k + 2 >= NBUF)
        lax.fori_loop(1, n_main // NBUF, main_body, 0)

        for g in range(n_main, n_chunks):
            nb = g % NBUF
            finish(nb, g)
            if g + NBUF < n_chunks:
                stage(nb, g + NBUF)
            if g + 2 < n_chunks:
                prep((g + 2) % NBUF, g + 2, g + 2 >= NBUF)
        for g in range(n_chunks - NBUF, n_chunks):
            out_wait(g % NBUF, g)

    return body(ids_flat, mask_flat, table)


def kernel(input_ids, input_mask, embedding_table):
    batch, hist = input_ids.shape
    ids_flat = input_ids.reshape(-1).astype(jnp.int32)
    mask_flat = input_mask.reshape(-1).astype(jnp.int32)
    info = plsc.get_sparse_core_info()
    out = _sc_lookup(
        ids_flat,
        mask_flat,
        embedding_table,
        num_cores=info.num_cores,
        num_subcores=info.num_subcores,
    )
    return out.reshape(batch, hist, EMBED_DIM)
